# baseline (speedup 1.0000x reference)
import functools
import math

import jax
import jax.numpy as jnp
from jax import lax
from jax.experimental import pallas as pl
from jax.experimental.pallas import tpu as pltpu
from jax.experimental.pallas import tpu_sc as plsc

N_SAMPLES = 1048576
N_BINS = 1024
X_MIN, X_MAX = -4.0, 4.0
SIGMA = (X_MAX - X_MIN) / N_BINS
DELTA = (X_MAX - X_MIN) / (N_BINS - 1)
RHO = DELTA / SIGMA
RHO2 = RHO * RHO
W = 4

NC, NS, L = 2, 16, 16
NW = NC * NS
CHUNK = N_SAMPLES // NW
NVEC = CHUNK // L
NBLK = N_BINS // L
UNROLL = 4

PAD = 16
PADW = N_BINS + 2 * PAD

SCALE = 1.0 / (N_SAMPLES * SIGMA * math.sqrt(2.0 * math.pi))
C_RATIO = [math.exp(-RHO2 * (k + 0.5)) for k in range(-W, W)]


def _sc_body(x_hbm, part_hbm, x_v, acc_v, part_v):
    wid = lax.axis_index("s") * NC + lax.axis_index("c")
    base = wid * CHUNK
    pltpu.sync_copy(x_hbm.at[pl.ds(base, CHUNK)], x_v)

    zero = jnp.zeros((L,), jnp.float32)
    rowbase = lax.iota(jnp.int32, L) * PADW + PAD

    def zero_blk(b, carry):
        for r in range(L):
            acc_v[pl.ds(pl.multiple_of(r * PADW + b * L, L), L)] = zero
        return carry

    lax.fori_loop(0, PADW // L, zero_blk, 0)

    def one_vec(i):
        xv = x_v[pl.ds(pl.multiple_of(i * L, L), L)]
        t = (xv - X_MIN) * (1.0 / DELTA)
        j0 = (t + 0.5).astype(jnp.int32)
        j0 = jnp.minimum(jnp.maximum(j0, -W - 1), N_BINS + W)
        u = t - j0.astype(jnp.float32)
        u = jnp.minimum(jnp.maximum(u, -1.0), 1.0)
        g = jnp.exp(RHO2 * u)
        w0 = u + W
        v = jnp.exp((-0.5 * RHO2) * (w0 * w0))
        jb = rowbase + j0
        for k in range(-W, W + 1):
            plsc.addupdate_scatter(acc_v, [jb + k], v)
            if k < W:
                v = v * (g * C_RATIO[k + W])

    def sample_blk(ii, carry):
        i0 = ii * UNROLL
        for s in range(UNROLL):
            one_vec(i0 + s)
        return carry

    lax.fori_loop(0, NVEC // UNROLL, sample_blk, 0)

    def reduce_blk(b, carry):
        tot = acc_v[pl.ds(pl.multiple_of(PAD + b * L, L), L)]
        for r in range(1, L):
            tot = tot + acc_v[pl.ds(pl.multiple_of(r * PADW + PAD + b * L, L), L)]
        part_v[pl.ds(pl.multiple_of(b * L, L), L)] = tot
        return carry

    lax.fori_loop(0, NBLK, reduce_blk, 0)
    pltpu.sync_copy(part_v, part_hbm.at[wid])


_sc_hist = functools.partial(
    pl.kernel,
    out_type=jax.ShapeDtypeStruct((NW, N_BINS), jnp.float32),
    mesh=plsc.VectorSubcoreMesh(core_axis_name="c", subcore_axis_name="s"),
    scratch_types=[
        pltpu.VMEM((CHUNK,), jnp.float32),
        pltpu.VMEM((L * PADW,), jnp.float32),
        pltpu.VMEM((N_BINS,), jnp.float32),
    ],
    compiler_params=pltpu.CompilerParams(needs_layout_passes=False),
)(_sc_body)


def _tc_reduce(p_ref, o_ref):
    o_ref[...] = jnp.sum(p_ref[...], axis=0, keepdims=True) * SCALE


@jax.jit
def kernel(x):
    partials = _sc_hist(x)
    hist = pl.pallas_call(
        _tc_reduce,
        out_shape=jax.ShapeDtypeStruct((1, N_BINS), jnp.float32),
    )(partials)
    return hist.reshape(N_BINS)


# speedup vs baseline: 1.8602x; 1.8602x over previous
import functools
import math

import jax
import jax.numpy as jnp
from jax import lax
from jax.experimental import pallas as pl
from jax.experimental.pallas import tpu as pltpu
from jax.experimental.pallas import tpu_sc as plsc

N_SAMPLES = 1048576
N_BINS = 1024
X_MIN, X_MAX = -4.0, 4.0
SIGMA = (X_MAX - X_MIN) / N_BINS
DELTA = (X_MAX - X_MIN) / (N_BINS - 1)
RHO = DELTA / SIGMA
RHO2 = RHO * RHO
W = 3

NC, NS, L = 2, 16, 16
NW = NC * NS
CHUNK = N_SAMPLES // NW
NVEC = CHUNK // L
NBLK = N_BINS // L
UNROLL = 4

PAD = 16
PADW = N_BINS + 2 * PAD

SCALE = 1.0 / (N_SAMPLES * SIGMA * math.sqrt(2.0 * math.pi))
C_TAP = [math.exp(-0.5 * RHO2 * k * k) for k in range(0, W + 1)]
T_LO = -(W + 1.4)
T_HI = N_BINS + W + 0.4


def _sc_body(x_hbm, part_hbm, x_v, acc_v, part_v):
    wid = lax.axis_index("s") * NC + lax.axis_index("c")
    base = wid * CHUNK
    pltpu.sync_copy(x_hbm.at[pl.ds(base, CHUNK)], x_v)

    zero = jnp.zeros((L,), jnp.float32)
    rowbase = lax.iota(jnp.int32, L) * PADW + PAD

    def zero_blk(b, carry):
        for r in range(L):
            acc_v[pl.ds(pl.multiple_of(r * PADW + b * L, L), L)] = zero
        return carry

    lax.fori_loop(0, PADW // L, zero_blk, 0)

    def sample_blk(ii, carry):
        i0 = ii * UNROLL
        taps = []
        for s in range(UNROLL):
            xv = x_v[pl.ds(pl.multiple_of((i0 + s) * L, L), L)]
            t = (xv - X_MIN) * (1.0 / DELTA)
            t = jnp.minimum(jnp.maximum(t, T_LO), T_HI)
            j0 = (t + 0.5).astype(jnp.int32)
            u = t - j0.astype(jnp.float32)
            a = jnp.exp((-0.5 * RHO2) * (u * u))
            b = jnp.exp(RHO2 * u)
            bi = jnp.exp((-RHO2) * u)
            b2 = b * b
            bi2 = bi * bi
            pw = {0: None, 1: b, 2: b2, 3: b2 * b, 4: b2 * b2,
                  -1: bi, -2: bi2, -3: bi2 * bi, -4: bi2 * bi2}
            ac = {k: a * C_TAP[k] for k in range(1, W + 1)}
            jb = rowbase + j0
            vals = [(k, a if k == 0 else ac[abs(k)] * pw[k])
                    for k in range(-W, W + 1)]
            taps.append((jb, vals))
        for jb, vals in taps:
            for k, v in vals:
                plsc.addupdate_scatter(acc_v, [jb + k], v)
        return carry

    lax.fori_loop(0, NVEC // UNROLL, sample_blk, 0)

    def reduce_blk(b, carry):
        tot = acc_v[pl.ds(pl.multiple_of(PAD + b * L, L), L)]
        for r in range(1, L):
            tot = tot + acc_v[pl.ds(pl.multiple_of(r * PADW + PAD + b * L, L), L)]
        part_v[pl.ds(pl.multiple_of(b * L, L), L)] = tot
        return carry

    lax.fori_loop(0, NBLK, reduce_blk, 0)
    pltpu.sync_copy(part_v, part_hbm.at[wid])


_sc_hist = functools.partial(
    pl.kernel,
    out_type=jax.ShapeDtypeStruct((NW, N_BINS), jnp.float32),
    mesh=plsc.VectorSubcoreMesh(core_axis_name="c", subcore_axis_name="s"),
    scratch_types=[
        pltpu.VMEM((CHUNK,), jnp.float32),
        pltpu.VMEM((L * PADW,), jnp.float32),
        pltpu.VMEM((N_BINS,), jnp.float32),
    ],
    compiler_params=pltpu.CompilerParams(needs_layout_passes=False),
)(_sc_body)


def _tc_reduce(p_ref, o_ref):
    o_ref[...] = jnp.sum(p_ref[...], axis=0, keepdims=True) * SCALE


@jax.jit
def kernel(x):
    partials = _sc_hist(x)
    hist = pl.pallas_call(
        _tc_reduce,
        out_shape=jax.ShapeDtypeStruct((1, N_BINS), jnp.float32),
    )(partials)
    return hist.reshape(N_BINS)
